# baseline (speedup 1.0000x reference)
import functools

import jax
import jax.numpy as jnp
from jax import lax
from jax.experimental import pallas as pl
from jax.experimental.pallas import tpu as pltpu
from jax.experimental.pallas import tpu_sc as plsc

B = 16
K = 500
KP = 512
H = 152
W = 272
HW = H * W
NW = 32
HALF = KP // 2
NCH = HALF // 16
TBL = 1024


def _sc_body(ids_hbm, ids2_hbm, index2_hbm, index_hbm, mask_hbm, wh_hbm,
             wh2_hbm, flow_hbm, pwh_hbm, out_hbm,
             ids_v, ids2_v, index2_v, index_v, mask_v, wh_v, wh2_v,
             idxg_v, gath_v, t1_v, t2_v, outv, sem_a, sem_b):
    cid = lax.axis_index("c")
    sid = lax.axis_index("s")
    wid = sid * 2 + cid
    b = wid // 2
    half = wid % 2
    hbase = half * HALF

    cps = [
        pltpu.async_copy(ids_hbm.at[b], ids_v.at[pl.ds(0, K)], sem_a),
        pltpu.async_copy(ids2_hbm.at[b], ids2_v.at[pl.ds(0, K)], sem_a),
        pltpu.async_copy(index2_hbm.at[b], index2_v.at[pl.ds(0, K)], sem_a),
        pltpu.async_copy(index_hbm.at[b], index_v.at[pl.ds(0, K)], sem_a),
        pltpu.async_copy(mask_hbm.at[b], mask_v.at[pl.ds(0, K)], sem_a),
        pltpu.async_copy(wh_hbm.at[b], wh_v.at[pl.ds(0, K)], sem_a),
        pltpu.async_copy(wh2_hbm.at[b], wh2_v.at[pl.ds(0, K)], sem_a),
    ]
    for cp in cps:
        cp.wait()

    iota = lax.iota(jnp.int32, 16)

    for k in range(NCH):
        hw = index_v[pl.ds(hbase + k * 16, 16)]
        hw = jnp.minimum(jnp.maximum(hw, 0), HW - 1)
        row = k // 8
        col = (k % 8) * 16
        for c in range(2):
            idxg_v[2 * c + row, pl.ds(col, 16)] = hw + (b * 2 + c) * HW
        for c in range(4):
            idxg_v[4 + 2 * c + row, pl.ds(col, 16)] = hw + (b * 4 + c) * HW

    gcs = [pltpu.async_copy(flow_hbm.at[idxg_v.at[r]], gath_v.at[r], sem_b)
           for r in range(4)]
    gcs += [pltpu.async_copy(pwh_hbm.at[idxg_v.at[r]], gath_v.at[r], sem_b)
            for r in range(4, 12)]

    big = jnp.full((16,), 1 << 20, jnp.int32)
    minus1 = jnp.full((16,), -1, jnp.int32)
    for j in range(TBL // 16):
        t1_v[pl.ds(j * 16, 16)] = big
        t2_v[pl.ds(j * 16, 16)] = minus1

    def shift_lanes(x, idx):
        dnums = lax.GatherDimensionNumbers(
            offset_dims=(), collapsed_slice_dims=(0,), start_index_map=(0,))
        return lax.gather(x, idx[:, None], dnums, (1,),
                          mode=lax.GatherScatterMode.PROMISE_IN_BOUNDS)

    shift_up = jnp.minimum(iota + 1, 15)
    shift_dn = jnp.maximum(iota - 1, 0)

    def chunk_keys(src_v, k):
        v = src_v[pl.ds(k * 16, 16)]
        ks = v * 16 + iota
        if (k + 1) * 16 > K:
            ks = jnp.where(iota < K - k * 16, ks, (TBL * 16) + iota)
        return ks

    for k in range(KP // 16 - 1, -1, -1):
        ks, xs = plsc.sort_key_val(chunk_keys(ids_v, k), iota + k * 16)
        vs = lax.shift_right_logical(ks, 4)
        winner = ((vs != shift_lanes(vs, shift_dn)) | (iota == 0)) & (vs < TBL)
        plsc.store_scatter(t1_v, [vs], xs, mask=winner)
    for k in range(KP // 16):
        ks, isrt = plsc.sort_key_val(chunk_keys(ids2_v, k), iota + k * 16)
        vs = lax.shift_right_logical(ks, 4)
        winner = ((vs != shift_lanes(vs, shift_up)) | (iota == 15)) & (vs < TBL)
        plsc.store_scatter(t2_v, [vs], isrt, mask=winner)

    for cp in gcs:
        cp.wait()

    zero = jnp.zeros((16,), jnp.float32)
    acc_xy = zero
    acc_wh = zero
    acc_m = zero
    for k in range(NCH):
        row = k // 8
        col = (k % 8) * 16
        xg = iota + (k * 16 + hbase)
        inb = xg < K
        v = jnp.bitwise_and(ids_v[pl.ds(hbase + k * 16, 16)], TBL - 1)
        t1 = plsc.load_gather(t1_v, [v])
        t2 = plsc.load_gather(t2_v, [v])
        recv = (v != 0) & (t1 == xg) & (t2 >= 0) & inb
        w = jnp.bitwise_and(t2, KP - 1)
        ix2 = plsc.load_gather(index2_v, [w])
        xc = (ix2 % W).astype(jnp.float32)
        yc = ix2.astype(jnp.float32) / float(W)
        rin0 = jnp.where(recv, xc, 0.0)
        rin1 = jnp.where(recv, yc, 0.0)
        m = mask_v[pl.ds(hbase + k * 16, 16)].astype(jnp.float32)
        m = jnp.where(inb, m, 0.0)
        f0 = gath_v[row, pl.ds(col, 16)]
        f1 = gath_v[2 + row, pl.ds(col, 16)]
        acc_xy = acc_xy + jnp.abs(f0 * m - rin0) + jnp.abs(f1 * m - rin1)
        for c in range(4):
            csplat = jnp.full((16,), c, jnp.int32)
            rw = jnp.where(recv, plsc.load_gather(wh2_v, [w, csplat]), 0.0)
            whc = jnp.where(inb, plsc.load_gather(wh_v, [xg, csplat]), 0.0)
            d = rw - whc
            term = jnp.where(d != -whc, d, 0.0) * m
            pw = gath_v[4 + 2 * c + row, pl.ds(col, 16)]
            acc_wh = acc_wh + jnp.abs(pw * m - term)
        acc_m = acc_m + m

    outv[0, :] = acc_xy
    outv[1, :] = acc_wh
    outv[2, :] = acc_m
    pltpu.sync_copy(outv, out_hbm.at[wid])


_sc_call = functools.partial(
    pl.kernel,
    out_type=jax.ShapeDtypeStruct((NW, 3, 16), jnp.float32),
    mesh=plsc.VectorSubcoreMesh(core_axis_name="c", subcore_axis_name="s"),
    compiler_params=pltpu.CompilerParams(needs_layout_passes=False, use_tc_tiling_on_sc=False),
    scratch_types=[
        pltpu.VMEM((KP,), jnp.int32),
        pltpu.VMEM((KP,), jnp.int32),
        pltpu.VMEM((KP,), jnp.int32),
        pltpu.VMEM((KP,), jnp.int32),
        pltpu.VMEM((KP,), jnp.int32),
        pltpu.VMEM((KP, 4), jnp.float32),
        pltpu.VMEM((KP, 4), jnp.float32),
        pltpu.VMEM((12, 128), jnp.int32),
        pltpu.VMEM((12, 128), jnp.float32),
        pltpu.VMEM((TBL,), jnp.int32),
        pltpu.VMEM((TBL,), jnp.int32),
        pltpu.VMEM((3, 16), jnp.float32),
        pltpu.SemaphoreType.DMA,
        pltpu.SemaphoreType.DMA,
    ],
)


def kernel(flow, p_wh, mask, index, ids, wh, index2, ids2, wh2):
    parts = _sc_call(_sc_body)(ids, ids2, index2, index, mask, wh, wh2,
                               flow.reshape(-1), p_wh.reshape(-1))
    s = parts.sum(axis=(0, 2))
    loss = s[0] / (2.0 * s[2] + 1e-4)
    wh_loss = s[1] / (4.0 * s[2] + 1e-4)
    return (loss, wh_loss)


# speedup vs baseline: 1.2915x; 1.2915x over previous
import functools

import jax
import jax.numpy as jnp
from jax import lax
from jax.experimental import pallas as pl
from jax.experimental.pallas import tpu as pltpu
from jax.experimental.pallas import tpu_sc as plsc

B = 16
K = 500
KP = 512
H = 152
W = 272
HW = H * W
NW = 32
HALF = KP // 2
NCH = HALF // 16
TBL = 1024


def _shift_lanes(x, idx):
    dnums = lax.GatherDimensionNumbers(
        offset_dims=(), collapsed_slice_dims=(0,), start_index_map=(0,))
    return lax.gather(x, idx[:, None], dnums, (1,),
                      mode=lax.GatherScatterMode.PROMISE_IN_BOUNDS)


def _sc_a_body(ints_hbm, idxg_out, t1_out, t2_out,
               ids_v, ids2_v, idxh_v, idxg_v, t1_v, t2_v, sem_a):
    cid = lax.axis_index("c")
    sid = lax.axis_index("s")
    wid = sid * 2 + cid
    b = wid // 2
    half = wid % 2
    hbase = half * HALF
    ib = b * 5

    cps = [
        pltpu.async_copy(ints_hbm.at[ib + 0], ids_v, sem_a),
        pltpu.async_copy(ints_hbm.at[ib + 1], ids2_v, sem_a),
        pltpu.async_copy(ints_hbm.at[ib + 3, pl.ds(hbase, HALF)], idxh_v,
                         sem_a),
    ]
    for cp in cps:
        cp.wait()

    for k in range(NCH):
        hw = idxh_v[pl.ds(k * 16, 16)]
        row = k // 8
        col = (k % 8) * 16
        for c in range(2):
            idxg_v[2 * c + row, pl.ds(col, 16)] = hw + (b * 2 + c) * HW
        for c in range(4):
            idxg_v[4 + 2 * c + row, pl.ds(col, 16)] = hw + (b * 4 + c) * HW

    big = jnp.full((16,), 1 << 20, jnp.int32)
    minus1 = jnp.full((16,), -1, jnp.int32)
    for j in range(TBL // 16):
        t1_v[pl.ds(j * 16, 16)] = big
        t2_v[pl.ds(j * 16, 16)] = minus1

    iota = lax.iota(jnp.int32, 16)
    shift_up = jnp.minimum(iota + 1, 15)
    shift_dn = jnp.maximum(iota - 1, 0)
    for k in range(KP // 16 - 1, -1, -1):
        xg = iota + k * 16
        v = ids_v[pl.ds(k * 16, 16)]
        ks, xs = plsc.sort_key_val(v * 16 + iota, xg)
        vs = lax.shift_right_logical(ks, 4)
        winner = (vs != _shift_lanes(vs, shift_dn)) | (iota == 0)
        plsc.store_scatter(t1_v, [vs], xs, mask=winner)
    for k in range(KP // 16):
        ig = iota + k * 16
        v = ids2_v[pl.ds(k * 16, 16)]
        ks, isrt = plsc.sort_key_val(v * 16 + iota, ig)
        vs = lax.shift_right_logical(ks, 4)
        winner = (vs != _shift_lanes(vs, shift_up)) | (iota == 15)
        plsc.store_scatter(t2_v, [vs], isrt, mask=winner)

    ocs = [
        pltpu.async_copy(idxg_v, idxg_out.at[wid], sem_a),
        pltpu.async_copy(t1_v, t1_out.at[wid], sem_a),
        pltpu.async_copy(t2_v, t2_out.at[wid], sem_a),
    ]
    for cp in ocs:
        cp.wait()


def _sc_b_body(ints_hbm, flts_hbm, flow_hbm, pwh_hbm, idxg_all, t1_all,
               t2_all, out_hbm,
               idsh_v, index2_v, wh2_v, maskh_v, whh_v, idxg_v, gath_v,
               t1_v, t2_v, outv, sem_a, sem_b):
    cid = lax.axis_index("c")
    sid = lax.axis_index("s")
    wid = sid * 2 + cid
    b = wid // 2
    half = wid % 2
    hbase = half * HALF
    ib = b * 5
    fb = b * 2

    pltpu.async_copy(idxg_all.at[wid], idxg_v, sem_a).wait()
    gcs = [pltpu.async_copy(flow_hbm.at[idxg_v.at[r]], gath_v.at[r], sem_b)
           for r in range(4)]
    gcs += [pltpu.async_copy(pwh_hbm.at[idxg_v.at[r]], gath_v.at[r], sem_b)
            for r in range(4, 12)]

    cps = [
        pltpu.async_copy(t1_all.at[wid], t1_v, sem_a),
        pltpu.async_copy(t2_all.at[wid], t2_v, sem_a),
        pltpu.async_copy(ints_hbm.at[ib + 0, pl.ds(hbase, HALF)], idsh_v,
                         sem_a),
        pltpu.async_copy(ints_hbm.at[ib + 2], index2_v, sem_a),
        pltpu.async_copy(flts_hbm.at[fb + 1], wh2_v, sem_a),
        pltpu.async_copy(ints_hbm.at[ib + 4, pl.ds(hbase, HALF)], maskh_v,
                         sem_a),
        pltpu.async_copy(flts_hbm.at[fb + 0, :, pl.ds(hbase, HALF)], whh_v,
                         sem_a),
    ]
    for cp in cps:
        cp.wait()
    for cp in gcs:
        cp.wait()

    iota = lax.iota(jnp.int32, 16)
    acc_xy = jnp.zeros((16,), jnp.float32)
    acc_wh = jnp.zeros((16,), jnp.float32)
    acc_m = jnp.zeros((16,), jnp.float32)
    for k in range(NCH):
        s = k * 16
        row = k // 8
        col = (k % 8) * 16
        v = idsh_v[pl.ds(s, 16)]
        t1 = plsc.load_gather(t1_v, [v])
        t2 = plsc.load_gather(t2_v, [v])
        xg = iota + (s + hbase)
        recv = (v != 0) & (t1 == xg) & (t2 >= 0) & (xg < K)
        w = jnp.maximum(t2, 0)
        ix2 = plsc.load_gather(index2_v, [w])
        xc = (ix2 % W).astype(jnp.float32)
        yc = ix2.astype(jnp.float32) / float(W)
        rin0 = jnp.where(recv, xc, 0.0)
        rin1 = jnp.where(recv, yc, 0.0)
        m = maskh_v[pl.ds(s, 16)].astype(jnp.float32)
        f0 = gath_v[row, pl.ds(col, 16)]
        f1 = gath_v[2 + row, pl.ds(col, 16)]
        acc_xy = acc_xy + jnp.abs(f0 * m - rin0) + jnp.abs(f1 * m - rin1)
        for c in range(4):
            rw = jnp.where(
                recv,
                plsc.load_gather(wh2_v, [jnp.full((16,), c, jnp.int32), w]),
                0.0)
            whc = whh_v[c, pl.ds(s, 16)]
            d = rw - whc
            term = jnp.where(d != -whc, d, 0.0) * m
            pw = gath_v[4 + 2 * c + row, pl.ds(col, 16)]
            acc_wh = acc_wh + jnp.abs(pw * m - term)
        acc_m = acc_m + m

    outv[0, :] = acc_xy
    outv[1, :] = acc_wh
    outv[2, :] = acc_m
    pltpu.sync_copy(outv, out_hbm.at[wid])


_MESH = plsc.VectorSubcoreMesh(core_axis_name="c", subcore_axis_name="s")

_sc_a = functools.partial(
    pl.kernel,
    out_type=(
        jax.ShapeDtypeStruct((NW, 12, 128), jnp.int32),
        jax.ShapeDtypeStruct((NW, TBL), jnp.int32),
        jax.ShapeDtypeStruct((NW, TBL), jnp.int32),
    ),
    mesh=_MESH,
    compiler_params=pltpu.CompilerParams(needs_layout_passes=False),
    scratch_types=[
        pltpu.VMEM((KP,), jnp.int32),
        pltpu.VMEM((KP,), jnp.int32),
        pltpu.VMEM((HALF,), jnp.int32),
        pltpu.VMEM((12, 128), jnp.int32),
        pltpu.VMEM((TBL,), jnp.int32),
        pltpu.VMEM((TBL,), jnp.int32),
        pltpu.SemaphoreType.DMA,
    ],
)

_sc_b = functools.partial(
    pl.kernel,
    out_type=jax.ShapeDtypeStruct((NW, 3, 16), jnp.float32),
    mesh=_MESH,
    compiler_params=pltpu.CompilerParams(needs_layout_passes=False),
    scratch_types=[
        pltpu.VMEM((HALF,), jnp.int32),
        pltpu.VMEM((KP,), jnp.int32),
        pltpu.VMEM((4, KP), jnp.float32),
        pltpu.VMEM((HALF,), jnp.int32),
        pltpu.VMEM((4, HALF), jnp.float32),
        pltpu.VMEM((12, 128), jnp.int32),
        pltpu.VMEM((12, 128), jnp.float32),
        pltpu.VMEM((TBL,), jnp.int32),
        pltpu.VMEM((TBL,), jnp.int32),
        pltpu.VMEM((3, 16), jnp.float32),
        pltpu.SemaphoreType.DMA,
        pltpu.SemaphoreType.DMA,
    ],
)


def kernel(flow, p_wh, mask, index, ids, wh, index2, ids2, wh2):
    ints = jnp.pad(jnp.stack([ids, ids2, index2, index, mask], axis=1),
                   ((0, 0), (0, 0), (0, KP - K))).reshape(B * 5, KP)
    flts = jnp.pad(jnp.stack([wh, wh2], axis=1).transpose(0, 1, 3, 2),
                   ((0, 0), (0, 0), (0, 0), (0, KP - K))).reshape(B * 2, 4, KP)
    flow_flat = flow.reshape(-1)
    pwh_flat = p_wh.reshape(-1)
    idxg_all, t1_all, t2_all = _sc_a(_sc_a_body)(ints)
    parts = _sc_b(_sc_b_body)(ints, flts, flow_flat, pwh_flat,
                              idxg_all, t1_all, t2_all)
    s = parts.sum(axis=(0, 2))
    loss = s[0] / (2.0 * s[2] + 1e-4)
    wh_loss = s[1] / (4.0 * s[2] + 1e-4)
    return (loss, wh_loss)


# speedup vs baseline: 1.6908x; 1.3092x over previous
import functools

import jax
import jax.numpy as jnp
from jax import lax
from jax.experimental import pallas as pl
from jax.experimental.pallas import tpu as pltpu
from jax.experimental.pallas import tpu_sc as plsc

B = 16
K = 500
KP = 512
H = 152
W = 272
HW = H * W
NW = 32
NCH = KP // 16
TBL = 1024


def _shift_lanes(x, idx):
    dnums = lax.GatherDimensionNumbers(
        offset_dims=(), collapsed_slice_dims=(0,), start_index_map=(0,))
    return lax.gather(x, idx[:, None], dnums, (1,),
                      mode=lax.GatherScatterMode.PROMISE_IN_BOUNDS)


def _sc_body(ints_hbm, flts_hbm, flow_hbm, pwh_hbm, out_hbm,
             ids_v, ids2_v, index_v, mask_v, index2_v, wh_v, wh2_v,
             pa_v, t1_v, t2_v, outv, sem_a, sem_b):
    cid = lax.axis_index("c")
    sid = lax.axis_index("s")
    wid = sid * 2 + cid
    b = wid // 2
    j = wid % 2
    ib = b * 5
    fb = b * 2

    cps = [
        pltpu.async_copy(ints_hbm.at[ib + 0], ids_v, sem_a),
        pltpu.async_copy(ints_hbm.at[ib + 1], ids2_v, sem_a),
        pltpu.async_copy(ints_hbm.at[ib + 2], index2_v, sem_a),
        pltpu.async_copy(ints_hbm.at[ib + 3], index_v, sem_a),
        pltpu.async_copy(ints_hbm.at[ib + 4], mask_v, sem_a),
        pltpu.async_copy(flts_hbm.at[fb + 0], wh_v, sem_a),
        pltpu.async_copy(flts_hbm.at[fb + 1], wh2_v, sem_a),
    ]
    for cp in cps:
        cp.wait()

    big = jnp.full((16,), 1 << 20, jnp.int32)
    minus1 = jnp.full((16,), -1, jnp.int32)
    for jj in range(TBL // 16):
        t1_v[pl.ds(jj * 16, 16)] = big
        t2_v[pl.ds(jj * 16, 16)] = minus1

    iota = lax.iota(jnp.int32, 16)
    shift_up = jnp.minimum(iota + 1, 15)
    shift_dn = jnp.maximum(iota - 1, 0)
    for k in range(NCH - 1, -1, -1):
        xg = iota + k * 16
        v = ids_v[pl.ds(k * 16, 16)]
        ks, xs = plsc.sort_key_val(v * 16 + iota, xg)
        vs = lax.shift_right_logical(ks, 4)
        winner = (vs != _shift_lanes(vs, shift_dn)) | (iota == 0)
        plsc.store_scatter(t1_v, [vs], xs, mask=winner)
    for k in range(NCH):
        ig = iota + k * 16
        v = ids2_v[pl.ds(k * 16, 16)]
        ks, isrt = plsc.sort_key_val(v * 16 + iota, ig)
        vs = lax.shift_right_logical(ks, 4)
        winner = (vs != _shift_lanes(vs, shift_up)) | (iota == 15)
        plsc.store_scatter(t2_v, [vs], isrt, mask=winner)

    def common_chunk(s):
        xg = iota + s
        v = ids_v[pl.ds(s, 16)]
        t1 = plsc.load_gather(t1_v, [v])
        t2 = plsc.load_gather(t2_v, [v])
        recv = (v != 0) & (t1 == xg) & (t2 >= 0) & (xg < K)
        w = jnp.maximum(t2, 0)
        m = mask_v[pl.ds(s, 16)].astype(jnp.float32)
        hw = index_v[pl.ds(s, 16)]
        hh = hw // W
        wc = hw % W
        return xg, recv, w, m, hh, wc

    def xy_pass(plane_v, c, acc0, count_m):
        def body(k, carry):
            acc, acc_m = carry
            s = k * 16
            xg, recv, w, m, hh, wc = common_chunk(s)
            f = plsc.load_gather(plane_v, [hh, wc])
            ix2 = plsc.load_gather(index2_v, [w])
            if c == 0:
                val = (ix2 % W).astype(jnp.float32)
            else:
                val = ix2.astype(jnp.float32) / float(W)
            rin = jnp.where(recv, val, 0.0)
            acc = acc + jnp.abs(f * m - rin)
            if count_m:
                acc_m = acc_m + m
            return (acc, acc_m)

        return lax.fori_loop(0, NCH, body,
                             (acc0, jnp.zeros((16,), jnp.float32)))

    def wh_pass(plane_v, c, acc0):
        csplat = jnp.full((16,), c, jnp.int32)

        def body(k, acc):
            s = k * 16
            xg, recv, w, m, hh, wc = common_chunk(s)
            pw = plsc.load_gather(plane_v, [hh, wc])
            rw = jnp.where(recv, plsc.load_gather(wh2_v, [csplat, w]), 0.0)
            whc = plsc.load_gather(wh_v, [csplat, xg])
            d = rw - whc
            term = jnp.where(d != -whc, d, 0.0) * m
            acc = acc + jnp.abs(pw * m - term)
            return acc

        return lax.fori_loop(0, NCH, body, acc0)

    zero = jnp.zeros((16,), jnp.float32)

    @pl.when(j == 0)
    def _j0():
        pltpu.async_copy(flow_hbm.at[b, 0], pa_v, sem_b).wait()
        acc_xy, acc_m = xy_pass(pa_v, 0, zero, True)
        pltpu.async_copy(flow_hbm.at[b, 1], pa_v, sem_b).wait()
        acc_xy, _ = xy_pass(pa_v, 1, acc_xy, False)
        pltpu.async_copy(pwh_hbm.at[b, 0], pa_v, sem_b).wait()
        acc_wh = wh_pass(pa_v, 0, zero)
        outv[0, :] = acc_xy
        outv[1, :] = acc_wh
        outv[2, :] = acc_m

    @pl.when(j == 1)
    def _j1():
        pltpu.async_copy(pwh_hbm.at[b, 1], pa_v, sem_b).wait()
        acc_wh = wh_pass(pa_v, 1, zero)
        pltpu.async_copy(pwh_hbm.at[b, 2], pa_v, sem_b).wait()
        acc_wh = wh_pass(pa_v, 2, acc_wh)
        pltpu.async_copy(pwh_hbm.at[b, 3], pa_v, sem_b).wait()
        acc_wh = wh_pass(pa_v, 3, acc_wh)
        outv[0, :] = zero
        outv[1, :] = acc_wh
        outv[2, :] = zero

    pltpu.sync_copy(outv, out_hbm.at[wid])


_sc_call = functools.partial(
    pl.kernel,
    out_type=jax.ShapeDtypeStruct((NW, 3, 16), jnp.float32),
    mesh=plsc.VectorSubcoreMesh(core_axis_name="c", subcore_axis_name="s"),
    compiler_params=pltpu.CompilerParams(needs_layout_passes=False),
    scratch_types=[
        pltpu.VMEM((KP,), jnp.int32),
        pltpu.VMEM((KP,), jnp.int32),
        pltpu.VMEM((KP,), jnp.int32),
        pltpu.VMEM((KP,), jnp.int32),
        pltpu.VMEM((KP,), jnp.int32),
        pltpu.VMEM((4, KP), jnp.float32),
        pltpu.VMEM((4, KP), jnp.float32),
        pltpu.VMEM((H, W), jnp.float32),
        pltpu.VMEM((TBL,), jnp.int32),
        pltpu.VMEM((TBL,), jnp.int32),
        pltpu.VMEM((3, 16), jnp.float32),
        pltpu.SemaphoreType.DMA,
        pltpu.SemaphoreType.DMA,
    ],
)


def kernel(flow, p_wh, mask, index, ids, wh, index2, ids2, wh2):
    ints = jnp.pad(jnp.stack([ids, ids2, index2, index, mask], axis=1),
                   ((0, 0), (0, 0), (0, KP - K))).reshape(B * 5, KP)
    flts = jnp.pad(jnp.stack([wh, wh2], axis=1).transpose(0, 1, 3, 2),
                   ((0, 0), (0, 0), (0, 0), (0, KP - K))).reshape(B * 2, 4, KP)
    parts = _sc_call(_sc_body)(ints, flts, flow, p_wh)
    s = parts.sum(axis=(0, 2))
    loss = s[0] / (2.0 * s[2] + 1e-4)
    wh_loss = s[1] / (4.0 * s[2] + 1e-4)
    return (loss, wh_loss)


# speedup vs baseline: 1.8628x; 1.1017x over previous
import functools

import jax
import jax.numpy as jnp
from jax import lax
from jax.experimental import pallas as pl
from jax.experimental.pallas import tpu as pltpu
from jax.experimental.pallas import tpu_sc as plsc

B = 16
K = 500
KP = 512
H = 152
W = 272
HW = H * W
NW = 32
NCH = KP // 16
TBL = 1024


def _shift_lanes(x, idx):
    dnums = lax.GatherDimensionNumbers(
        offset_dims=(), collapsed_slice_dims=(0,), start_index_map=(0,))
    return lax.gather(x, idx[:, None], dnums, (1,),
                      mode=lax.GatherScatterMode.PROMISE_IN_BOUNDS)


def _sc_body(ints_hbm, flts_hbm, flow_hbm, pwh_hbm, out_hbm,
             ids_v, ids2_v, index_v, mask_v, index2_v, wh_v, wh2_v,
             pa_v, pb_v, t1_v, t2_v, outv, sem_a, sem_b, sem_c):
    cid = lax.axis_index("c")
    sid = lax.axis_index("s")
    wid = sid * 2 + cid
    b = wid // 2
    j = wid % 2
    ib = b * 5
    fb = b * 2

    @pl.when(j == 0)
    def _fire0():
        pltpu.async_copy(flow_hbm.at[b, 0], pa_v, sem_b)
        pltpu.async_copy(flow_hbm.at[b, 1], pb_v, sem_c)

    @pl.when(j == 1)
    def _fire1():
        pltpu.async_copy(pwh_hbm.at[b, 1], pa_v, sem_b)
        pltpu.async_copy(pwh_hbm.at[b, 2], pb_v, sem_c)

    cps = [
        pltpu.async_copy(ints_hbm.at[ib + 0], ids_v, sem_a),
        pltpu.async_copy(ints_hbm.at[ib + 1], ids2_v, sem_a),
        pltpu.async_copy(ints_hbm.at[ib + 2], index2_v, sem_a),
        pltpu.async_copy(ints_hbm.at[ib + 3], index_v, sem_a),
        pltpu.async_copy(ints_hbm.at[ib + 4], mask_v, sem_a),
        pltpu.async_copy(flts_hbm.at[fb + 0], wh_v, sem_a),
        pltpu.async_copy(flts_hbm.at[fb + 1], wh2_v, sem_a),
    ]
    for cp in cps:
        cp.wait()

    iota = lax.iota(jnp.int32, 16)
    big = jnp.full((16,), 1 << 20, jnp.int32)
    minus1 = jnp.full((16,), -1, jnp.int32)

    def init_body(jj, carry):
        t1_v[pl.ds(jj * 16, 16)] = big
        t2_v[pl.ds(jj * 16, 16)] = minus1
        return carry

    lax.fori_loop(0, TBL // 16, init_body, 0)

    shift_up = jnp.minimum(iota + 1, 15)
    shift_dn = jnp.maximum(iota - 1, 0)

    def t1_body(i, carry):
        s = (NCH - 1 - i) * 16
        xg = iota + s
        v = ids_v[pl.ds(s, 16)]
        ks, xs = plsc.sort_key_val(v * 16 + iota, xg)
        vs = lax.shift_right_logical(ks, 4)
        winner = (vs != _shift_lanes(vs, shift_dn)) | (iota == 0)
        plsc.store_scatter(t1_v, [vs], xs, mask=winner)
        return carry

    lax.fori_loop(0, NCH, t1_body, 0)

    def t2_body(k, carry):
        s = k * 16
        ig = iota + s
        v = ids2_v[pl.ds(s, 16)]
        ks, isrt = plsc.sort_key_val(v * 16 + iota, ig)
        vs = lax.shift_right_logical(ks, 4)
        winner = (vs != _shift_lanes(vs, shift_up)) | (iota == 15)
        plsc.store_scatter(t2_v, [vs], isrt, mask=winner)
        return carry

    lax.fori_loop(0, NCH, t2_body, 0)

    def common_chunk(s):
        xg = iota + s
        v = ids_v[pl.ds(s, 16)]
        t1 = plsc.load_gather(t1_v, [v])
        t2 = plsc.load_gather(t2_v, [v])
        recv = (v != 0) & (t1 == xg) & (t2 >= 0) & (xg < K)
        w = jnp.maximum(t2, 0)
        m = mask_v[pl.ds(s, 16)].astype(jnp.float32)
        hw = index_v[pl.ds(s, 16)]
        hh = hw // W
        wc = hw % W
        return xg, recv, w, m, hh, wc

    def xy_pass(plane_v, c, acc0, count_m):
        def body(k, carry):
            acc, acc_m = carry
            s = k * 16
            xg, recv, w, m, hh, wc = common_chunk(s)
            f = plsc.load_gather(plane_v, [hh, wc])
            ix2 = plsc.load_gather(index2_v, [w])
            if c == 0:
                val = (ix2 % W).astype(jnp.float32)
            else:
                val = ix2.astype(jnp.float32) / float(W)
            rin = jnp.where(recv, val, 0.0)
            acc = acc + jnp.abs(f * m - rin)
            if count_m:
                acc_m = acc_m + m
            return (acc, acc_m)

        return lax.fori_loop(0, NCH, body,
                             (acc0, jnp.zeros((16,), jnp.float32)))

    def wh_pass(plane_v, c, acc0):
        csplat = jnp.full((16,), c, jnp.int32)

        def body(k, acc):
            s = k * 16
            xg, recv, w, m, hh, wc = common_chunk(s)
            pw = plsc.load_gather(plane_v, [hh, wc])
            rw = jnp.where(recv, plsc.load_gather(wh2_v, [csplat, w]), 0.0)
            whc = plsc.load_gather(wh_v, [csplat, xg])
            d = rw - whc
            term = jnp.where(d != -whc, d, 0.0) * m
            acc = acc + jnp.abs(pw * m - term)
            return acc

        return lax.fori_loop(0, NCH, body, acc0)

    zero = jnp.zeros((16,), jnp.float32)

    @pl.when(j == 0)
    def _j0():
        pltpu.make_async_copy(flow_hbm.at[b, 0], pa_v, sem_b).wait()
        acc_xy, acc_m = xy_pass(pa_v, 0, zero, True)
        c2 = pltpu.async_copy(pwh_hbm.at[b, 0], pa_v, sem_b)
        pltpu.make_async_copy(flow_hbm.at[b, 1], pb_v, sem_c).wait()
        acc_xy, _ = xy_pass(pb_v, 1, acc_xy, False)
        c2.wait()
        acc_wh = wh_pass(pa_v, 0, zero)
        outv[0, :] = acc_xy
        outv[1, :] = acc_wh
        outv[2, :] = acc_m

    @pl.when(j == 1)
    def _j1():
        pltpu.make_async_copy(pwh_hbm.at[b, 1], pa_v, sem_b).wait()
        acc_wh = wh_pass(pa_v, 1, zero)
        c2 = pltpu.async_copy(pwh_hbm.at[b, 3], pa_v, sem_b)
        pltpu.make_async_copy(pwh_hbm.at[b, 2], pb_v, sem_c).wait()
        acc_wh = wh_pass(pb_v, 2, acc_wh)
        c2.wait()
        acc_wh = wh_pass(pa_v, 3, acc_wh)
        outv[0, :] = zero
        outv[1, :] = acc_wh
        outv[2, :] = zero

    pltpu.sync_copy(outv, out_hbm.at[wid])


_sc_call = functools.partial(
    pl.kernel,
    out_type=jax.ShapeDtypeStruct((NW, 3, 16), jnp.float32),
    mesh=plsc.VectorSubcoreMesh(core_axis_name="c", subcore_axis_name="s"),
    compiler_params=pltpu.CompilerParams(needs_layout_passes=False),
    scratch_types=[
        pltpu.VMEM((KP,), jnp.int32),
        pltpu.VMEM((KP,), jnp.int32),
        pltpu.VMEM((KP,), jnp.int32),
        pltpu.VMEM((KP,), jnp.int32),
        pltpu.VMEM((KP,), jnp.int32),
        pltpu.VMEM((4, KP), jnp.float32),
        pltpu.VMEM((4, KP), jnp.float32),
        pltpu.VMEM((H, W), jnp.float32),
        pltpu.VMEM((H, W), jnp.float32),
        pltpu.VMEM((TBL,), jnp.int32),
        pltpu.VMEM((TBL,), jnp.int32),
        pltpu.VMEM((3, 16), jnp.float32),
        pltpu.SemaphoreType.DMA,
        pltpu.SemaphoreType.DMA,
        pltpu.SemaphoreType.DMA,
    ],
)


def kernel(flow, p_wh, mask, index, ids, wh, index2, ids2, wh2):
    ints = jnp.pad(jnp.stack([ids, ids2, index2, index, mask], axis=1),
                   ((0, 0), (0, 0), (0, KP - K))).reshape(B * 5, KP)
    flts = jnp.pad(jnp.stack([wh, wh2], axis=1).transpose(0, 1, 3, 2),
                   ((0, 0), (0, 0), (0, 0), (0, KP - K))).reshape(B * 2, 4, KP)
    parts = _sc_call(_sc_body)(ints, flts, flow, p_wh)
    s = parts.sum(axis=(0, 2))
    loss = s[0] / (2.0 * s[2] + 1e-4)
    wh_loss = s[1] / (4.0 * s[2] + 1e-4)
    return (loss, wh_loss)
